# baseline (speedup 1.0000x reference)
import functools

import jax
import jax.numpy as jnp
from jax import lax
from jax.experimental import pallas as pl
from jax.experimental.pallas import tpu as pltpu
from jax.experimental.pallas import tpu_sc as plsc

VOCAB_SIZE = 1000000
D = 128
BATCH = 4096
SEQ = 200
B_TOTAL = BATCH * SEQ
NC, NS = 2, 16
NW = NC * NS
PER_W = B_TOTAL // NW
CH = 128
NCH = PER_W // CH
NV = 4
NSP = 2

LO, HI = 2, 194

_mesh = plsc.VectorSubcoreMesh(core_axis_name="c", subcore_axis_name="s")


@functools.partial(
    pl.kernel,
    out_type=jax.ShapeDtypeStruct((NW * NCH, CH, D), jnp.float32),
    mesh=_mesh,
    scratch_types=[
        pltpu.VMEM((NCH, CH), jnp.int32),
        pltpu.VMEM((NV, CH, D), jnp.float32),
        pltpu.VMEM_SHARED((NS, NSP, CH, D), jnp.float32),
    ] + [pltpu.SemaphoreType.DMA] * (NV + 2 * NSP),
)
def _sc_gather(table_hbm, idx_hbm, out_hbm, idx_v, rows_v, sp, *sems):
    gsem = sems[:NV]
    csem = sems[NV:NV + NSP]
    osem = sems[NV + NSP:]
    sid = lax.axis_index("s")
    wid = sid * NC + lax.axis_index("c")
    pltpu.sync_copy(idx_hbm.at[wid], idx_v)

    def g_desc(j, s):
        return pltpu.make_async_copy(
            table_hbm.at[idx_v.at[j]], rows_v.at[s], gsem[s])

    def c_desc(s, t):
        return pltpu.make_async_copy(rows_v.at[s], sp.at[sid, t], csem[t])

    def o_desc(j, t):
        return pltpu.make_async_copy(
            sp.at[sid, t], out_hbm.at[wid * NCH + j], osem[t])

    g_desc(0, 0).start()
    g_desc(1, 1).start()
    g_desc(2, 2).start()
    g_desc(0, 0).wait()
    c_desc(0, 0).start()
    g_desc(3, 3).start()
    g_desc(1, 1).wait()
    c_desc(1, 1).start()
    c_desc(0, 0).wait()
    o_desc(0, 0).start()
    g_desc(4, 0).start()

    @pl.loop(LO, HI, step=NV)
    def _(g):
        for b in range(NV):
            j = g + b
            s = (b + 2) % NV
            t = b % NSP
            g_desc(j, s).wait()
            o_desc(j - 2, t).wait()
            c_desc(s, t).start()
            c_desc((s + 3) % NV, (t + 1) % NSP).wait()
            o_desc(j - 1, (t + 1) % NSP).start()
            g_desc(j + 3, (s + 3) % NV).start()

    for j in range(HI, NCH):
        s = j % NV
        t = j % NSP
        g_desc(j, s).wait()
        o_desc(j - 2, t).wait()
        c_desc(s, t).start()
        c_desc((s + 3) % NV, (t + 1) % NSP).wait()
        o_desc(j - 1, (t + 1) % NSP).start()
        if j + 3 < NCH:
            g_desc(j + 3, (s + 3) % NV).start()
    c_desc((NCH - 1) % NV, (NCH - 1) % NSP).wait()
    o_desc(NCH - 1, (NCH - 1) % NSP).start()
    o_desc(NCH - 2, (NCH - 2) % NSP).wait()
    o_desc(NCH - 1, (NCH - 1) % NSP).wait()


def kernel(token_ids, weight):
    idx = token_ids.reshape(NW, NCH, CH).astype(jnp.int32)
    out = _sc_gather(weight, idx)
    return out.reshape(BATCH, SEQ, D)


# speedup vs baseline: 1.0022x; 1.0022x over previous
import functools

import jax
import jax.numpy as jnp
from jax import lax
from jax.experimental import pallas as pl
from jax.experimental.pallas import tpu as pltpu
from jax.experimental.pallas import tpu_sc as plsc

VOCAB_SIZE = 1000000
D = 128
BATCH = 4096
SEQ = 200
B_TOTAL = BATCH * SEQ
NC, NS = 2, 16
NW = NC * NS
PER_W = B_TOTAL // NW
CH = 128
NCH = PER_W // CH
NR = 3

LO, HI = 2, 197

_mesh = plsc.VectorSubcoreMesh(core_axis_name="c", subcore_axis_name="s")


@functools.partial(
    pl.kernel,
    out_type=jax.ShapeDtypeStruct((NW * NCH, CH, D), jnp.float32),
    mesh=_mesh,
    scratch_types=[
        pltpu.VMEM((NCH, CH), jnp.int32),
        pltpu.VMEM((NR, CH, D), jnp.float32),
        pltpu.VMEM_SHARED((NS, NR, CH, D), jnp.float32),
    ] + [pltpu.SemaphoreType.DMA] * (3 * NR),
)
def _sc_gather(table_hbm, idx_hbm, out_hbm, idx_v, rows_v, sp, *sems):
    gsem = sems[:NR]
    csem = sems[NR:2 * NR]
    osem = sems[2 * NR:]
    sid = lax.axis_index("s")
    wid = sid * NC + lax.axis_index("c")
    pltpu.sync_copy(idx_hbm.at[wid], idx_v)

    def g_desc(j, s):
        return pltpu.make_async_copy(
            table_hbm.at[idx_v.at[j]], rows_v.at[s], gsem[s])

    def c_desc(s):
        return pltpu.make_async_copy(rows_v.at[s], sp.at[sid, s], csem[s])

    def o_desc(j, s):
        return pltpu.make_async_copy(
            sp.at[sid, s], out_hbm.at[wid * NCH + j], osem[s])

    g_desc(0, 0).start()
    g_desc(1, 1).start()
    g_desc(0, 0).wait()
    c_desc(0).start()
    g_desc(2, 2).start()
    g_desc(1, 1).wait()
    c_desc(1).start()
    c_desc(0).wait()
    o_desc(0, 0).start()
    g_desc(3, 0).start()

    @pl.loop(LO, HI, step=NR)
    def _(g):
        for b in range(NR):
            j = g + b
            s = (b + 2) % NR
            g_desc(j, s).wait()
            c_desc(s).start()
            c_desc((s + 2) % NR).wait()
            o_desc(j - 1, (s + 2) % NR).start()
            o_desc(j - 2, (s + 1) % NR).wait()
            g_desc(j + 2, (s + 2) % NR).start()

    for j in range(HI, NCH):
        s = j % NR
        g_desc(j, s).wait()
        c_desc(s).start()
        c_desc((s + 2) % NR).wait()
        o_desc(j - 1, (s + 2) % NR).start()
        o_desc(j - 2, (s + 1) % NR).wait()
        if j + 2 < NCH:
            g_desc(j + 2, (s + 2) % NR).start()
    c_desc((NCH - 1) % NR).wait()
    o_desc(NCH - 1, (NCH - 1) % NR).start()
    o_desc(NCH - 2, (NCH - 2) % NR).wait()
    o_desc(NCH - 1, (NCH - 1) % NR).wait()


def kernel(token_ids, weight):
    idx = token_ids.reshape(NW, NCH, CH).astype(jnp.int32)
    out = _sc_gather(weight, idx)
    return out.reshape(BATCH, SEQ, D)
